# baseline (speedup 1.0000x reference)
import functools

import jax
import jax.numpy as jnp
from jax import lax
from jax.experimental import pallas as pl
from jax.experimental.pallas import tpu as pltpu
from jax.experimental.pallas import tpu_sc as plsc

N = 100000
D = 128
H = 128
NC = 16
ALPHA = 0.2

NUM_WORKERS = 32
CHUNK = 128
NUM_SC_CORES = 2
K0 = 25
NPS = NUM_WORKERS * K0 * CHUNK
RING = 4

PRE_ROWS = 12800
POST_ROWS = 12800


def _tc_pre(features, C, W, V):
  def body(f_ref, c_ref, w_ref, v_ref, o_ref, a_ref, b_ref):
    @pl.when(pl.program_id(0) == 0)
    def _fold():
      cw1 = jnp.dot(c_ref[...], w_ref[:H, :],
                    preferred_element_type=jnp.float32)
      cw2 = jnp.dot(c_ref[...], w_ref[H:, :],
                    preferred_element_type=jnp.float32)
      a_ref[...] = jnp.dot(cw1, v_ref[...],
                           preferred_element_type=jnp.float32)
      b_ref[...] = jnp.dot(cw2, v_ref[...],
                           preferred_element_type=jnp.float32)

    f = f_ref[...]
    ya = jnp.dot(f, a_ref[...], preferred_element_type=jnp.float32)
    yb = jnp.dot(f, b_ref[...], preferred_element_type=jnp.float32)
    o_ref[...] = jnp.concatenate(
        [ya, yb, jnp.zeros((ya.shape[0], D - 2 * NC), jnp.float32)], axis=1)

  return pl.pallas_call(
      body,
      grid=(pl.cdiv(N, PRE_ROWS),),
      in_specs=[
          pl.BlockSpec((PRE_ROWS, D), lambda i: (i, 0)),
          pl.BlockSpec((D, H), lambda i: (0, 0)),
          pl.BlockSpec((2 * H, H), lambda i: (0, 0)),
          pl.BlockSpec((H, NC), lambda i: (0, 0)),
      ],
      out_specs=pl.BlockSpec((PRE_ROWS, D), lambda i: (i, 0)),
      out_shape=jax.ShapeDtypeStruct((N, D), jnp.float32),
      scratch_shapes=[
          pltpu.VMEM((H, NC), jnp.float32),
          pltpu.VMEM((H, NC), jnp.float32),
      ],
  )(features, C, W, V)


def _sc_gather(table, i1, i2):
  mesh = plsc.VectorSubcoreMesh(core_axis_name="c", subcore_axis_name="s",
                                num_cores=NUM_SC_CORES)

  @functools.partial(
      pl.kernel,
      out_type=(
          jax.ShapeDtypeStruct((NPS // 8, D), jnp.float32),
          jax.ShapeDtypeStruct((NPS // 8, D), jnp.float32),
      ),
      mesh=mesh,
      compiler_params=pltpu.CompilerParams(use_tc_tiling_on_sc=False),
      scratch_types=[
          pltpu.VMEM((K0 * CHUNK,), jnp.int32),
          pltpu.VMEM((K0 * CHUNK,), jnp.int32),
          pltpu.VMEM((RING, CHUNK, NC), jnp.float32),
          pltpu.VMEM((RING, CHUNK, NC), jnp.float32),
          pltpu.VMEM((RING, CHUNK // 8, D), jnp.float32),
          pltpu.VMEM((RING, CHUNK // 8, D), jnp.float32),
          pltpu.SemaphoreType.DMA((RING,)),
          pltpu.SemaphoreType.DMA((RING,)),
          pltpu.SemaphoreType.DMA((RING,)),
          pltpu.SemaphoreType.DMA((RING,)),
      ],
  )
  def gather_kernel(t_hbm, i1_hbm, i2_hbm, g1_hbm, g2_hbm,
                    idx1_v, idx2_v, buf1, buf2, pk1, pk2,
                    gs1, gs2, ws1, ws2):
    cid = lax.axis_index("c")
    sid = lax.axis_index("s")
    wid = cid * 16 + sid
    kcount = K0
    cstart = wid * K0

    def fire_gather(k, b):
      pltpu.async_copy(t_hbm.at[idx1_v.at[pl.ds(k * CHUNK, CHUNK)]],
                       buf1.at[b], gs1.at[b])
      pltpu.async_copy(t_hbm.at[idx2_v.at[pl.ds(k * CHUNK, CHUNK)]],
                       buf2.at[b], gs2.at[b])

    row0 = pl.multiple_of(cstart * CHUNK, CHUNK)
    pltpu.sync_copy(i1_hbm.at[pl.ds(row0, K0 * CHUNK)], idx1_v)
    pltpu.sync_copy(i2_hbm.at[pl.ds(row0, K0 * CHUNK)], idx2_v)
    for b in range(RING):
      fire_gather(b, b)

    def wait_write(b):
      pltpu.make_async_copy(pk1.at[b], g1_hbm.at[pl.ds(0, CHUNK // 8)],
                            ws1.at[b]).wait()
      pltpu.make_async_copy(pk2.at[b], g2_hbm.at[pl.ds(0, CHUNK // 8)],
                            ws2.at[b]).wait()

    def repack(b):
      def row(r, carry):
        q = r // 8
        j = r - q * 8
        pk1[b, q, pl.dslice(j * NC, NC)] = buf1[b, r, :]
        pk2[b, q, pl.dslice(j * NC, NC)] = buf2[b, r, :]
        return carry

      lax.fori_loop(0, CHUNK, row, 0)

    def body(j, carry):
      b = lax.rem(j, RING)
      off = pl.multiple_of((cstart + j) * (CHUNK // 8), CHUNK // 8)
      pltpu.make_async_copy(t_hbm.at[pl.ds(0, CHUNK)], buf1.at[b],
                            gs1.at[b]).wait()
      pltpu.make_async_copy(t_hbm.at[pl.ds(0, CHUNK)], buf2.at[b],
                            gs2.at[b]).wait()

      @pl.when(j >= RING)
      def _drain_prev():
        wait_write(b)

      repack(b)
      pltpu.async_copy(pk1.at[b], g1_hbm.at[pl.ds(off, CHUNK // 8)],
                       ws1.at[b])
      pltpu.async_copy(pk2.at[b], g2_hbm.at[pl.ds(off, CHUNK // 8)],
                       ws2.at[b])

      @pl.when(j + RING < kcount)
      def _refill():
        fire_gather(j + RING, b)

      return carry

    lax.fori_loop(0, kcount, body, 0)

    for b in range(RING):
      wait_write(b)

  return gather_kernel(table, i1, i2)


POST_BLK = 128


def _tc_post(g1p, g2p):
  def body(g1_ref, g2_ref, o_ref):
    r = g1_ref[...] + g2_ref[...]
    r = jnp.where(r >= 0, r, ALPHA * r)
    e = jnp.exp(r)
    lane = jax.lax.broadcasted_iota(jnp.int32, (D, D), 0) // NC
    lane_t = jax.lax.broadcasted_iota(jnp.int32, (D, D), 1) // NC
    bd = (lane == lane_t).astype(jnp.float32)
    seg = jnp.dot(e, bd, preferred_element_type=jnp.float32)
    res = r - jnp.log(seg)

    ncols = 8 * POST_BLK
    q_iota = jax.lax.broadcasted_iota(jnp.int32, (POST_BLK, ncols), 0)
    m_iota = jax.lax.broadcasted_iota(jnp.int32, (POST_BLK, ncols), 1)
    out = jnp.zeros((NC, ncols), jnp.float32)
    for j in range(8):
      qj = jnp.logical_and(m_iota // 8 == q_iota,
                           m_iota % 8 == j).astype(jnp.float32)
      out = out + jnp.dot(res[:, NC * j:NC * (j + 1)].T, qj,
                          preferred_element_type=jnp.float32)
    o_ref[...] = out

  return pl.pallas_call(
      body,
      grid=(pl.cdiv(N, 8 * POST_BLK),),
      in_specs=[
          pl.BlockSpec((POST_BLK, D), lambda i: (i, 0)),
          pl.BlockSpec((POST_BLK, D), lambda i: (i, 0)),
      ],
      out_specs=pl.BlockSpec((NC, 8 * POST_BLK), lambda i: (0, i)),
      out_shape=jax.ShapeDtypeStruct((NC, N), jnp.float32),
  )(g1p, g2p)


def kernel(features, C, W, V, n1, n2):
  def pack(idx, sub):
    pad = jnp.arange(NPS - N, dtype=jnp.int32)
    full = jnp.concatenate([idx.astype(jnp.int32), pad])
    return full * 8 + sub

  table = _tc_pre(features, C, W, V).reshape(8 * N, NC)
  g1p, g2p = _sc_gather(table, pack(n1, 0), pack(n2, 1))
  return _tc_post(g1p, g2p).T


# speedup vs baseline: 1.0057x; 1.0057x over previous
import functools

import jax
import jax.numpy as jnp
from jax import lax
from jax.experimental import pallas as pl
from jax.experimental.pallas import tpu as pltpu
from jax.experimental.pallas import tpu_sc as plsc

N = 100000
D = 128
H = 128
NC = 16
ALPHA = 0.2

NUM_WORKERS = 32
CHUNK = 128
NUM_SC_CORES = 2
K0 = 25
NPS = NUM_WORKERS * K0 * CHUNK
RING = 4

PRE_ROWS = 12800
POST_ROWS = 12800


def _tc_pre(features, C, W, V):
  def body(f_ref, c_ref, w_ref, v_ref, o_ref, a_ref, b_ref):
    @pl.when(pl.program_id(0) == 0)
    def _fold():
      cw1 = jnp.dot(c_ref[...], w_ref[:H, :],
                    preferred_element_type=jnp.float32)
      cw2 = jnp.dot(c_ref[...], w_ref[H:, :],
                    preferred_element_type=jnp.float32)
      a_ref[...] = jnp.dot(cw1, v_ref[...],
                           preferred_element_type=jnp.float32)
      b_ref[...] = jnp.dot(cw2, v_ref[...],
                           preferred_element_type=jnp.float32)

    f = f_ref[...]
    ya = jnp.dot(f, a_ref[...], preferred_element_type=jnp.float32)
    yb = jnp.dot(f, b_ref[...], preferred_element_type=jnp.float32)
    o_ref[...] = jnp.concatenate(
        [ya, yb, jnp.zeros((ya.shape[0], D - 2 * NC), jnp.float32)], axis=1)

  return pl.pallas_call(
      body,
      grid=(pl.cdiv(N, PRE_ROWS),),
      in_specs=[
          pl.BlockSpec((PRE_ROWS, D), lambda i: (i, 0)),
          pl.BlockSpec((D, H), lambda i: (0, 0)),
          pl.BlockSpec((2 * H, H), lambda i: (0, 0)),
          pl.BlockSpec((H, NC), lambda i: (0, 0)),
      ],
      out_specs=pl.BlockSpec((PRE_ROWS, D), lambda i: (i, 0)),
      out_shape=jax.ShapeDtypeStruct((N, D), jnp.float32),
      scratch_shapes=[
          pltpu.VMEM((H, NC), jnp.float32),
          pltpu.VMEM((H, NC), jnp.float32),
      ],
  )(features, C, W, V)


def _sc_gather(table, i1, i2):
  mesh = plsc.VectorSubcoreMesh(core_axis_name="c", subcore_axis_name="s",
                                num_cores=NUM_SC_CORES)

  @functools.partial(
      pl.kernel,
      out_type=(
          jax.ShapeDtypeStruct((NPS // 8, D), jnp.float32),
          jax.ShapeDtypeStruct((NPS // 8, D), jnp.float32),
      ),
      mesh=mesh,
      compiler_params=pltpu.CompilerParams(use_tc_tiling_on_sc=False),
      scratch_types=[
          pltpu.VMEM((K0 * CHUNK,), jnp.int32),
          pltpu.VMEM((K0 * CHUNK,), jnp.int32),
          pltpu.VMEM((RING, CHUNK, NC), jnp.float32),
          pltpu.VMEM((RING, CHUNK, NC), jnp.float32),
          pltpu.VMEM((RING, CHUNK // 8, D), jnp.float32),
          pltpu.VMEM((RING, CHUNK // 8, D), jnp.float32),
          pltpu.SemaphoreType.DMA((RING,)),
          pltpu.SemaphoreType.DMA((RING,)),
          pltpu.SemaphoreType.DMA((RING,)),
          pltpu.SemaphoreType.DMA((RING,)),
      ],
  )
  def gather_kernel(t_hbm, i1_hbm, i2_hbm, g1_hbm, g2_hbm,
                    idx1_v, idx2_v, buf1, buf2, pk1, pk2,
                    gs1, gs2, ws1, ws2):
    cid = lax.axis_index("c")
    sid = lax.axis_index("s")
    wid = cid * 16 + sid
    kcount = K0
    cstart = wid * K0

    def fire_gather(k, b):
      pltpu.async_copy(t_hbm.at[idx1_v.at[pl.ds(k * CHUNK, CHUNK)]],
                       buf1.at[b], gs1.at[b])
      pltpu.async_copy(t_hbm.at[idx2_v.at[pl.ds(k * CHUNK, CHUNK)]],
                       buf2.at[b], gs2.at[b])

    row0 = pl.multiple_of(cstart * CHUNK, CHUNK)
    pltpu.sync_copy(i1_hbm.at[pl.ds(row0, K0 * CHUNK)], idx1_v)
    pltpu.sync_copy(i2_hbm.at[pl.ds(row0, K0 * CHUNK)], idx2_v)
    for b in range(RING):
      fire_gather(b, b)

    def wait_write(b):
      pltpu.make_async_copy(pk1.at[b], g1_hbm.at[pl.ds(0, CHUNK // 8)],
                            ws1.at[b]).wait()
      pltpu.make_async_copy(pk2.at[b], g2_hbm.at[pl.ds(0, CHUNK // 8)],
                            ws2.at[b]).wait()

    def repack(b):
      def row(r, carry):
        q = r // 8
        j = r - q * 8
        pk1[b, q, pl.dslice(j * NC, NC)] = buf1[b, r, :]
        pk2[b, q, pl.dslice(j * NC, NC)] = buf2[b, r, :]
        return carry

      lax.fori_loop(0, CHUNK, row, 0)

    def body(j, carry):
      b = lax.rem(j, RING)
      off = pl.multiple_of((cstart + j) * (CHUNK // 8), CHUNK // 8)
      pltpu.make_async_copy(t_hbm.at[pl.ds(0, CHUNK)], buf1.at[b],
                            gs1.at[b]).wait()
      pltpu.make_async_copy(t_hbm.at[pl.ds(0, CHUNK)], buf2.at[b],
                            gs2.at[b]).wait()

      @pl.when(j >= RING)
      def _drain_prev():
        wait_write(b)

      repack(b)
      pltpu.async_copy(pk1.at[b], g1_hbm.at[pl.ds(off, CHUNK // 8)],
                       ws1.at[b])
      pltpu.async_copy(pk2.at[b], g2_hbm.at[pl.ds(off, CHUNK // 8)],
                       ws2.at[b])

      @pl.when(j + RING < kcount)
      def _refill():
        fire_gather(j + RING, b)

      return carry

    lax.fori_loop(0, kcount, body, 0)

    for b in range(RING):
      wait_write(b)

  return gather_kernel(table, i1, i2)


POST_BLK = 128


def _tc_post(g1p, g2p):
  ncols = 8 * POST_BLK

  def body(g1_ref, g2_ref, o_ref, bd_ref, q_ref):
    @pl.when(pl.program_id(0) == 0)
    def _build_consts():
      lane = jax.lax.broadcasted_iota(jnp.int32, (D, D), 0) // NC
      lane_t = jax.lax.broadcasted_iota(jnp.int32, (D, D), 1) // NC
      bd_ref[...] = (lane == lane_t).astype(jnp.float32)
      q_iota = jax.lax.broadcasted_iota(jnp.int32, (POST_BLK, ncols), 0)
      m_iota = jax.lax.broadcasted_iota(jnp.int32, (POST_BLK, ncols), 1)
      for j in range(8):
        q_ref[j] = jnp.logical_and(m_iota // 8 == q_iota,
                                   m_iota % 8 == j).astype(jnp.float32)

    r = g1_ref[...] + g2_ref[...]
    r = jnp.where(r >= 0, r, ALPHA * r)
    e = jnp.exp(r)
    seg = jnp.dot(e, bd_ref[...], preferred_element_type=jnp.float32)
    res = r - jnp.log(seg)

    out = jnp.zeros((NC, ncols), jnp.float32)
    for j in range(8):
      out = out + jnp.dot(res[:, NC * j:NC * (j + 1)].T, q_ref[j],
                          preferred_element_type=jnp.float32)
    o_ref[...] = out

  return pl.pallas_call(
      body,
      grid=(pl.cdiv(N, 8 * POST_BLK),),
      in_specs=[
          pl.BlockSpec((POST_BLK, D), lambda i: (i, 0)),
          pl.BlockSpec((POST_BLK, D), lambda i: (i, 0)),
      ],
      out_specs=pl.BlockSpec((NC, 8 * POST_BLK), lambda i: (0, i)),
      out_shape=jax.ShapeDtypeStruct((NC, N), jnp.float32),
      scratch_shapes=[
          pltpu.VMEM((D, D), jnp.float32),
          pltpu.VMEM((8, POST_BLK, ncols), jnp.float32),
      ],
  )(g1p, g2p)


def kernel(features, C, W, V, n1, n2):
  def pack(idx, sub):
    pad = jnp.arange(NPS - N, dtype=jnp.int32)
    full = jnp.concatenate([idx.astype(jnp.int32), pad])
    return full * 8 + sub

  table = _tc_pre(features, C, W, V).reshape(8 * N, NC)
  g1p, g2p = _sc_gather(table, pack(n1, 0), pack(n2, 1))
  return _tc_post(g1p, g2p).T


# speedup vs baseline: 1.3713x; 1.3635x over previous
import functools

import jax
import jax.numpy as jnp
from jax import lax
from jax.experimental import pallas as pl
from jax.experimental.pallas import tpu as pltpu
from jax.experimental.pallas import tpu_sc as plsc

N = 100000
D = 128
H = 128
NC = 16
ALPHA = 0.2

NUM_WORKERS = 32
CHUNK = 128
NUM_SC_CORES = 2
K0 = 25
NPS = NUM_WORKERS * K0 * CHUNK
RING = 4

PRE_ROWS = 12800
POST_ROWS = 12800


def _tc_pre(features, C, W, V):
  def body(f_ref, c_ref, w_ref, v_ref, o_ref, a_ref, b_ref):
    @pl.when(pl.program_id(0) == 0)
    def _fold():
      cw1 = jnp.dot(c_ref[...], w_ref[:H, :],
                    preferred_element_type=jnp.float32)
      cw2 = jnp.dot(c_ref[...], w_ref[H:, :],
                    preferred_element_type=jnp.float32)
      a_ref[...] = jnp.dot(cw1, v_ref[...],
                           preferred_element_type=jnp.float32)
      b_ref[...] = jnp.dot(cw2, v_ref[...],
                           preferred_element_type=jnp.float32)

    f = f_ref[...]
    ya = jnp.dot(f, a_ref[...], preferred_element_type=jnp.float32)
    yb = jnp.dot(f, b_ref[...], preferred_element_type=jnp.float32)
    o_ref[...] = jnp.concatenate(
        [ya, yb, jnp.zeros((ya.shape[0], D - 2 * NC), jnp.float32)], axis=1)

  return pl.pallas_call(
      body,
      grid=(pl.cdiv(N, PRE_ROWS),),
      in_specs=[
          pl.BlockSpec((PRE_ROWS, D), lambda i: (i, 0)),
          pl.BlockSpec((D, H), lambda i: (0, 0)),
          pl.BlockSpec((2 * H, H), lambda i: (0, 0)),
          pl.BlockSpec((H, NC), lambda i: (0, 0)),
      ],
      out_specs=pl.BlockSpec((PRE_ROWS, D), lambda i: (i, 0)),
      out_shape=jax.ShapeDtypeStruct((N, D), jnp.float32),
      scratch_shapes=[
          pltpu.VMEM((H, NC), jnp.float32),
          pltpu.VMEM((H, NC), jnp.float32),
      ],
  )(features, C, W, V)


def _sc_gather(table, i1, i2):
  mesh = plsc.VectorSubcoreMesh(core_axis_name="c", subcore_axis_name="s",
                                num_cores=NUM_SC_CORES)

  @functools.partial(
      pl.kernel,
      out_type=(
          jax.ShapeDtypeStruct((NPS // 8, D), jnp.float32),
          jax.ShapeDtypeStruct((NPS // 8, D), jnp.float32),
      ),
      mesh=mesh,
      compiler_params=pltpu.CompilerParams(use_tc_tiling_on_sc=False),
      scratch_types=[
          pltpu.VMEM((K0 * CHUNK,), jnp.int32),
          pltpu.VMEM((K0 * CHUNK,), jnp.int32),
          pltpu.VMEM((RING, CHUNK, NC), jnp.float32),
          pltpu.VMEM((RING, CHUNK, NC), jnp.float32),
          pltpu.VMEM((RING, CHUNK // 8, D), jnp.float32),
          pltpu.VMEM((RING, CHUNK // 8, D), jnp.float32),
          pltpu.SemaphoreType.DMA((RING,)),
          pltpu.SemaphoreType.DMA((RING,)),
          pltpu.SemaphoreType.DMA((RING,)),
          pltpu.SemaphoreType.DMA((RING,)),
      ],
  )
  def gather_kernel(t_hbm, i1_hbm, i2_hbm, g1_hbm, g2_hbm,
                    idx1_v, idx2_v, buf1, buf2, pk1, pk2,
                    gs1, gs2, ws1, ws2):
    cid = lax.axis_index("c")
    sid = lax.axis_index("s")
    wid = cid * 16 + sid
    kcount = K0
    cstart = wid * K0

    def fire_gather(k, b):
      pltpu.async_copy(t_hbm.at[idx1_v.at[pl.ds(k * CHUNK, CHUNK)]],
                       buf1.at[b], gs1.at[b])
      pltpu.async_copy(t_hbm.at[idx2_v.at[pl.ds(k * CHUNK, CHUNK)]],
                       buf2.at[b], gs2.at[b])

    row0 = pl.multiple_of(cstart * CHUNK, CHUNK)
    pltpu.sync_copy(i1_hbm.at[pl.ds(row0, K0 * CHUNK)], idx1_v)
    pltpu.sync_copy(i2_hbm.at[pl.ds(row0, K0 * CHUNK)], idx2_v)
    for b in range(RING):
      fire_gather(b, b)

    def wait_write(b):
      pltpu.make_async_copy(pk1.at[b], g1_hbm.at[pl.ds(0, CHUNK // 8)],
                            ws1.at[b]).wait()
      pltpu.make_async_copy(pk2.at[b], g2_hbm.at[pl.ds(0, CHUNK // 8)],
                            ws2.at[b]).wait()

    def repack(b):
      def row(r, carry):
        q = r // 8
        j = r - q * 8
        pk1[b, q, pl.dslice(j * NC, NC)] = buf1[b, r, :]
        pk2[b, q, pl.dslice(j * NC, NC)] = buf2[b, r, :]
        return carry

      lax.fori_loop(0, CHUNK, row, 0)

    def body(j, carry):
      b = lax.rem(j, RING)
      off = pl.multiple_of((cstart + j) * (CHUNK // 8), CHUNK // 8)
      pltpu.make_async_copy(t_hbm.at[pl.ds(0, CHUNK)], buf1.at[b],
                            gs1.at[b]).wait()
      pltpu.make_async_copy(t_hbm.at[pl.ds(0, CHUNK)], buf2.at[b],
                            gs2.at[b]).wait()

      @pl.when(j >= RING)
      def _drain_prev():
        wait_write(b)

      repack(b)
      pltpu.async_copy(pk1.at[b], g1_hbm.at[pl.ds(off, CHUNK // 8)],
                       ws1.at[b])
      pltpu.async_copy(pk2.at[b], g2_hbm.at[pl.ds(off, CHUNK // 8)],
                       ws2.at[b])

      @pl.when(j + RING < kcount)
      def _refill():
        fire_gather(j + RING, b)

      return carry

    lax.fori_loop(0, kcount, body, 0)

    for b in range(RING):
      wait_write(b)

  return gather_kernel(table, i1, i2)


POST_BLK = 1024


def _tc_post(g1p, g2p):
  sub = 128
  nsub = POST_BLK // sub
  subcols = 8 * sub

  def body(g1_ref, g2_ref, o_ref, bd_ref, q_ref):
    @pl.when(pl.program_id(0) == 0)
    def _build_consts():
      lane = jax.lax.broadcasted_iota(jnp.int32, (D, D), 0) // NC
      lane_t = jax.lax.broadcasted_iota(jnp.int32, (D, D), 1) // NC
      bd_ref[...] = (lane == lane_t).astype(jnp.float32)
      q_iota = jax.lax.broadcasted_iota(jnp.int32, (sub, subcols), 0)
      m_iota = jax.lax.broadcasted_iota(jnp.int32, (sub, subcols), 1)
      for j in range(8):
        q_ref[j] = jnp.logical_and(m_iota // 8 == q_iota,
                                   m_iota % 8 == j).astype(jnp.float32)

    r = g1_ref[...] + g2_ref[...]
    r = jnp.where(r >= 0, r, ALPHA * r)
    e = jnp.exp(r)
    seg = jnp.dot(e, bd_ref[...], preferred_element_type=jnp.float32)
    res = r - jnp.log(seg)

    for s in range(nsub):
      rsub = res[sub * s:sub * (s + 1), :]
      out = jnp.zeros((NC, subcols), jnp.float32)
      for j in range(8):
        out = out + jnp.dot(rsub[:, NC * j:NC * (j + 1)].T, q_ref[j],
                            preferred_element_type=jnp.float32)
      o_ref[:, subcols * s:subcols * (s + 1)] = out

  return pl.pallas_call(
      body,
      grid=(pl.cdiv(N, 8 * POST_BLK),),
      in_specs=[
          pl.BlockSpec((POST_BLK, D), lambda i: (i, 0)),
          pl.BlockSpec((POST_BLK, D), lambda i: (i, 0)),
      ],
      out_specs=pl.BlockSpec((NC, 8 * POST_BLK), lambda i: (0, i)),
      out_shape=jax.ShapeDtypeStruct((NC, N), jnp.float32),
      scratch_shapes=[
          pltpu.VMEM((D, D), jnp.float32),
          pltpu.VMEM((8, sub, subcols), jnp.float32),
      ],
  )(g1p, g2p)


def kernel(features, C, W, V, n1, n2):
  def pack(idx, sub):
    pad = jnp.arange(NPS - N, dtype=jnp.int32)
    full = jnp.concatenate([idx.astype(jnp.int32), pad])
    return full * 8 + sub

  table = _tc_pre(features, C, W, V).reshape(8 * N, NC)
  g1p, g2p = _sc_gather(table, pack(n1, 0), pack(n2, 1))
  return _tc_post(g1p, g2p).T


# speedup vs baseline: 1.4967x; 1.0914x over previous
import functools

import jax
import jax.numpy as jnp
from jax import lax
from jax.experimental import pallas as pl
from jax.experimental.pallas import tpu as pltpu
from jax.experimental.pallas import tpu_sc as plsc

N = 100000
D = 128
H = 128
NC = 16
ALPHA = 0.2

NUM_WORKERS = 32
CHUNK = 128
NUM_SC_CORES = 2
K0 = 25
NPS = NUM_WORKERS * K0 * CHUNK
RING = 4

PRE_ROWS = 12800
POST_ROWS = 12800


def _tc_pre(features, C, W, V):
  def body(f_ref, c_ref, w_ref, v_ref, o_ref, a_ref, b_ref):
    @pl.when(pl.program_id(0) == 0)
    def _fold():
      cw1 = jnp.dot(c_ref[...], w_ref[:H, :],
                    preferred_element_type=jnp.float32)
      cw2 = jnp.dot(c_ref[...], w_ref[H:, :],
                    preferred_element_type=jnp.float32)
      a_ref[...] = jnp.dot(cw1, v_ref[...],
                           preferred_element_type=jnp.float32)
      b_ref[...] = jnp.dot(cw2, v_ref[...],
                           preferred_element_type=jnp.float32)

    f = f_ref[...]
    ya = jnp.dot(f, a_ref[...], preferred_element_type=jnp.float32)
    yb = jnp.dot(f, b_ref[...], preferred_element_type=jnp.float32)
    o_ref[...] = jnp.concatenate(
        [ya, yb, jnp.zeros((ya.shape[0], D - 2 * NC), jnp.float32)], axis=1)

  return pl.pallas_call(
      body,
      grid=(pl.cdiv(N, PRE_ROWS),),
      in_specs=[
          pl.BlockSpec((PRE_ROWS, D), lambda i: (i, 0)),
          pl.BlockSpec((D, H), lambda i: (0, 0)),
          pl.BlockSpec((2 * H, H), lambda i: (0, 0)),
          pl.BlockSpec((H, NC), lambda i: (0, 0)),
      ],
      out_specs=pl.BlockSpec((PRE_ROWS, D), lambda i: (i, 0)),
      out_shape=jax.ShapeDtypeStruct((N, D), jnp.float32),
      scratch_shapes=[
          pltpu.VMEM((H, NC), jnp.float32),
          pltpu.VMEM((H, NC), jnp.float32),
      ],
  )(features, C, W, V)


def _sc_gather(table, i1, i2):
  mesh = plsc.VectorSubcoreMesh(core_axis_name="c", subcore_axis_name="s",
                                num_cores=NUM_SC_CORES)

  @functools.partial(
      pl.kernel,
      out_type=(
          jax.ShapeDtypeStruct((NPS // 8, D), jnp.float32),
          jax.ShapeDtypeStruct((NPS // 8, D), jnp.float32),
      ),
      mesh=mesh,
      compiler_params=pltpu.CompilerParams(use_tc_tiling_on_sc=False),
      scratch_types=[
          pltpu.VMEM((K0 * CHUNK,), jnp.int32),
          pltpu.VMEM((K0 * CHUNK,), jnp.int32),
          pltpu.VMEM((RING, CHUNK, NC), jnp.float32),
          pltpu.VMEM((RING, CHUNK, NC), jnp.float32),
          pltpu.VMEM((RING, CHUNK // 8, D), jnp.float32),
          pltpu.VMEM((RING, CHUNK // 8, D), jnp.float32),
          pltpu.SemaphoreType.DMA((RING,)),
          pltpu.SemaphoreType.DMA((RING,)),
          pltpu.SemaphoreType.DMA((RING,)),
          pltpu.SemaphoreType.DMA((RING,)),
      ],
  )
  def gather_kernel(t_hbm, i1_hbm, i2_hbm, g1_hbm, g2_hbm,
                    idx1_v, idx2_v, buf1, buf2, pk1, pk2,
                    gs1, gs2, ws1, ws2):
    cid = lax.axis_index("c")
    sid = lax.axis_index("s")
    wid = cid * 16 + sid
    kcount = K0
    cstart = wid * K0

    def fire_gather(k, b):
      pltpu.async_copy(t_hbm.at[idx1_v.at[pl.ds(k * CHUNK, CHUNK)]],
                       buf1.at[b], gs1.at[b])
      pltpu.async_copy(t_hbm.at[idx2_v.at[pl.ds(k * CHUNK, CHUNK)]],
                       buf2.at[b], gs2.at[b])

    row0 = pl.multiple_of(cstart * CHUNK, CHUNK)
    pltpu.sync_copy(i1_hbm.at[pl.ds(row0, K0 * CHUNK)], idx1_v)
    pltpu.sync_copy(i2_hbm.at[pl.ds(row0, K0 * CHUNK)], idx2_v)
    for b in range(RING):
      fire_gather(b, b)

    def wait_write(b):
      pltpu.make_async_copy(pk1.at[b], g1_hbm.at[pl.ds(0, CHUNK // 8)],
                            ws1.at[b]).wait()
      pltpu.make_async_copy(pk2.at[b], g2_hbm.at[pl.ds(0, CHUNK // 8)],
                            ws2.at[b]).wait()

    def repack(b):
      def row(r, carry):
        q = r // 8
        j = r - q * 8
        pk1[b, q, pl.dslice(j * NC, NC)] = buf1[b, r, :]
        pk2[b, q, pl.dslice(j * NC, NC)] = buf2[b, r, :]
        return carry

      lax.fori_loop(0, CHUNK, row, 0)

    def body(j, carry):
      b = lax.rem(j, RING)
      off = pl.multiple_of((cstart + j) * (CHUNK // 8), CHUNK // 8)
      pltpu.make_async_copy(t_hbm.at[pl.ds(0, CHUNK)], buf1.at[b],
                            gs1.at[b]).wait()
      pltpu.make_async_copy(t_hbm.at[pl.ds(0, CHUNK)], buf2.at[b],
                            gs2.at[b]).wait()

      @pl.when(j >= RING)
      def _drain_prev():
        wait_write(b)

      repack(b)
      pltpu.async_copy(pk1.at[b], g1_hbm.at[pl.ds(off, CHUNK // 8)],
                       ws1.at[b])
      pltpu.async_copy(pk2.at[b], g2_hbm.at[pl.ds(off, CHUNK // 8)],
                       ws2.at[b])

      @pl.when(j + RING < kcount)
      def _refill():
        fire_gather(j + RING, b)

      return carry

    lax.fori_loop(0, kcount, body, 0)

    for b in range(RING):
      wait_write(b)

  return gather_kernel(table, i1, i2)


POST_BLK = 1024


def _tc_post(g1p, g2p):
  sub = 128
  nsub = POST_BLK // sub
  subcols = 8 * sub

  def body(g1_ref, g2_ref, o_ref, bd_ref, q_ref):
    @pl.when(pl.program_id(0) == 0)
    def _build_consts():
      lane = jax.lax.broadcasted_iota(jnp.int32, (D, D), 0) // NC
      lane_t = jax.lax.broadcasted_iota(jnp.int32, (D, D), 1) // NC
      bd_ref[...] = (lane == lane_t).astype(jnp.float32)
      q_iota = jax.lax.broadcasted_iota(jnp.int32, (sub, subcols), 0)
      m_iota = jax.lax.broadcasted_iota(jnp.int32, (sub, subcols), 1)
      for j in range(8):
        q_ref[j] = jnp.logical_and(m_iota // 8 == q_iota,
                                   m_iota % 8 == j).astype(jnp.float32)

    r = g1_ref[...] + g2_ref[...]
    r = jnp.where(r >= 0, r, ALPHA * r)
    e = jnp.exp(r)
    seg = jnp.dot(e, bd_ref[...], preferred_element_type=jnp.float32)
    res = r - jnp.log(seg)

    acc = [jnp.zeros((NC, subcols), jnp.float32) for _ in range(nsub)]
    for j in range(8):
      lj = jnp.concatenate(
          [res[sub * s:sub * (s + 1), NC * j:NC * (j + 1)].T
           for s in range(nsub)], axis=0)
      pj = jnp.dot(lj, q_ref[j], preferred_element_type=jnp.float32)
      for s in range(nsub):
        acc[s] = acc[s] + pj[NC * s:NC * (s + 1), :]
    for s in range(nsub):
      o_ref[:, subcols * s:subcols * (s + 1)] = acc[s]

  return pl.pallas_call(
      body,
      grid=(pl.cdiv(N, 8 * POST_BLK),),
      in_specs=[
          pl.BlockSpec((POST_BLK, D), lambda i: (i, 0)),
          pl.BlockSpec((POST_BLK, D), lambda i: (i, 0)),
      ],
      out_specs=pl.BlockSpec((NC, 8 * POST_BLK), lambda i: (0, i)),
      out_shape=jax.ShapeDtypeStruct((NC, N), jnp.float32),
      scratch_shapes=[
          pltpu.VMEM((D, D), jnp.float32),
          pltpu.VMEM((8, sub, subcols), jnp.float32),
      ],
  )(g1p, g2p)


def kernel(features, C, W, V, n1, n2):
  def pack(idx, sub):
    pad = jnp.arange(NPS - N, dtype=jnp.int32)
    full = jnp.concatenate([idx.astype(jnp.int32), pad])
    return full * 8 + sub

  table = _tc_pre(features, C, W, V).reshape(8 * N, NC)
  g1p, g2p = _sc_gather(table, pack(n1, 0), pack(n2, 1))
  return _tc_post(g1p, g2p).T
